# baseline (speedup 1.0000x reference)
import functools

import jax
import jax.numpy as jnp
import numpy as np
from jax import lax
from jax.experimental import pallas as pl
from jax.experimental.pallas import tpu as pltpu
from jax.experimental.pallas import tpu_sc as plsc


def _hilbert_flat(n: int) -> np.ndarray:
    d = np.arange(n * n, dtype=np.int64)
    x = np.zeros_like(d)
    y = np.zeros_like(d)
    t = d.copy()
    s = 1
    while s < n:
        rx = 1 & (t // 2)
        ry = 1 & (t ^ rx)
        swap = ry == 0
        flip = swap & (rx == 1)
        xf = np.where(flip, s - 1 - x, x)
        yf = np.where(flip, s - 1 - y, y)
        xn = np.where(swap, yf, xf)
        yn = np.where(swap, xf, yf)
        x = xn + s * rx
        y = yn + s * ry
        t = t // 4
        s *= 2
    return x * n + y


@functools.cache
def _build(B, H, W, C):
    n_pix = H * W
    SQ = 16
    SEG = SQ * C
    n_sq = n_pix // (SQ * SQ)
    n_items = B * n_sq

    info = plsc.get_sparse_core_info()
    NW = info.num_cores * info.num_subcores
    NC = info.num_cores
    per_w = n_items // NW
    sq_per_w = n_sq // (NW // B) if NW >= B else n_sq

    mesh = plsc.VectorSubcoreMesh(core_axis_name="c", subcore_axis_name="s")

    @functools.partial(
        pl.kernel,
        mesh=mesh,
        out_type=jax.ShapeDtypeStruct((B * n_pix, C), jnp.float32),
        compiler_params=pltpu.CompilerParams(use_tc_tiling_on_sc=False),
        scratch_types=[
            pltpu.VMEM((per_w * SQ,), jnp.int32),
            pltpu.VMEM((per_w * SQ * SQ,), jnp.int32),
            pltpu.VMEM((SQ, SEG), jnp.float32),
            pltpu.VMEM((SQ * SQ, C), jnp.float32),
            pltpu.SemaphoreType.DMA,
        ],
    )
    def gather_kernel(seg_hbm, segtab_hbm, lidx_hbm, out_hbm,
                      segidx_v, lidx_v, staged, out_buf, gsem):
        wid = lax.axis_index("s") * NC + lax.axis_index("c")
        b = wid // (NW // B)
        s0 = (wid % (NW // B)) * sq_per_w
        pltpu.sync_copy(segtab_hbm.at[pl.ds(s0 * SQ, per_w * SQ)], segidx_v)
        pltpu.sync_copy(lidx_hbm.at[pl.ds(s0 * SQ * SQ, per_w * SQ * SQ)],
                        lidx_v)

        def item_body(k, _):
            seg_vec = segidx_v[pl.ds(k * SQ, SQ)] + b * (n_pix // SQ)
            pltpu.async_copy(seg_hbm.at[seg_vec], staged, gsem).wait()

            kbase = k * SQ * SQ

            def row_group_body(g, _):
                lvec = lidx_v[pl.ds(kbase + g * 16, 16)]
                rb = g * 16
                for r16 in range(16):
                    off = lvec[r16]
                    si = lax.shift_right_logical(off, 10)
                    so = lax.bitwise_and(off, SEG - 1)
                    for c in range(C // 16):
                        out_buf[rb + r16, pl.ds(c * 16, 16)] = (
                            staged[si, pl.ds(so + c * 16, 16)])
                return 0

            lax.fori_loop(0, SQ * SQ // 16, row_group_body, 0)

            g = wid * per_w + k
            pltpu.sync_copy(out_buf, out_hbm.at[pl.ds(g * SQ * SQ, SQ * SQ)])
            return 0

        lax.fori_loop(0, per_w, item_body, 0)

    flat = _hilbert_flat(H)
    x = flat // W
    y = flat % W
    runs = flat.reshape(n_sq, SQ * SQ)
    xs = x.reshape(n_sq, SQ * SQ)
    ys = y.reshape(n_sq, SQ * SQ)
    X = (xs.min(axis=1) // SQ) * SQ
    Y = (ys.min(axis=1) // SQ) * SQ
    seg_tab = ((X[:, None] + np.arange(SQ)[None, :]) * (W // SQ)
               + Y[:, None] // SQ).reshape(-1).astype(np.int32)
    lidx = ((xs - X[:, None]) * SQ + (ys - Y[:, None])) * C
    lidx_tab = lidx.reshape(-1).astype(np.int32)
    return gather_kernel, jnp.asarray(seg_tab), jnp.asarray(lidx_tab)


def kernel(inputs):
    B, H, W, C = inputs.shape
    gather_kernel, seg_tab, lidx_tab = _build(B, H, W, C)
    seg_view = inputs.reshape(B * H * (W // 16), 16 * C)
    out = gather_kernel(seg_view, seg_tab, lidx_tab)
    return out.reshape(B, 1, H * W, C)
